# baseline (speedup 1.0000x reference)
import jax
import jax.numpy as jnp
from jax import lax
from jax.experimental import pallas as pl
from jax.experimental.pallas import tpu as pltpu
from jax.experimental.pallas import tpu_sc as plsc

D_MODEL = 2048
NUM_EXPERTS = 8
TOP_K = 2
NUM_TASKS = 64
D_TASK_EMBED = 64
D_FFN = 32
EF = NUM_EXPERTS * D_FFN

TOK_BLK = 512

SC_CORES = 2
SC_SUBCORES = 16
SC_WORKERS = SC_CORES * SC_SUBCORES
SC_LANES = 16


def _stage1_body(x_ref, w1_ref, b1_ref, temb_ref, gwt_ref, gb_ref,
                 h_ref, s1_ref, s2_ref, te_ref):
    x = x_ref[...]
    s1_ref[...] = jnp.full((1, 1, 128), jnp.sum(x), dtype=jnp.float32)
    s2_ref[...] = jnp.full((1, 1, 128), jnp.sum(x * x), dtype=jnp.float32)
    h = jnp.dot(x.astype(jnp.bfloat16), w1_ref[...],
                preferred_element_type=jnp.float32)
    h = h + b1_ref[...]
    h_ref[...] = (h * jax.nn.sigmoid(h)).astype(jnp.bfloat16)
    te_ref[...] = jnp.dot(temb_ref[...].astype(jnp.bfloat16),
                          gwt_ref[...].astype(jnp.bfloat16),
                          preferred_element_type=jnp.float32) + gb_ref[...]


def _stage2_body(x_ref, mean_ref, std_ref, te_ref, gwx_ref, lg_ref):
    i = pl.program_id(0)
    b = i // (pl.num_programs(0) // mean_ref.shape[0])
    mean_row = mean_ref[pl.ds(b, 1), :][:, :1]
    std_row = std_ref[pl.ds(b, 1), :][:, :1]
    te_row = te_ref[pl.ds(b, 1), :]
    xn = (x_ref[...] - mean_row) / std_row
    lg_ref[...] = jnp.dot(xn.astype(jnp.bfloat16),
                          gwx_ref[...].astype(jnp.bfloat16),
                          preferred_element_type=jnp.float32) + te_row


def _route_body(lg_hbm, w_hbm, idxt_hbm, cnt_hbm, lg_v, w_v, i1_v, i2_v, cnt_v):
    tok_per_w = lg_hbm.shape[0] // SC_WORKERS
    n_grp = tok_per_w // SC_LANES
    wid = lax.axis_index("s") * SC_CORES + lax.axis_index("c")
    base = wid * tok_per_w
    pltpu.sync_copy(lg_hbm.at[pl.ds(base, tok_per_w), :], lg_v)

    lane = lax.iota(jnp.int32, SC_LANES)
    neg_inf = jnp.full((SC_LANES,), -jnp.inf, jnp.float32)
    cnt = [jnp.zeros((SC_LANES,), jnp.float32) for _ in range(NUM_EXPERTS)]

    for j in range(n_grp):
        row = jnp.full((SC_LANES,), j * SC_LANES, jnp.int32) + lane
        v = [plsc.load_gather(lg_v, [row, jnp.full((SC_LANES,), e, jnp.int32)])
             for e in range(NUM_EXPERTS)]
        m1 = v[0]
        for e in range(1, NUM_EXPERTS):
            m1 = jnp.maximum(m1, v[e])
        i1 = jnp.full((SC_LANES,), NUM_EXPERTS, jnp.int32)
        for e in reversed(range(NUM_EXPERTS)):
            i1 = jnp.where(v[e] == m1, jnp.full((SC_LANES,), e, jnp.int32), i1)
        l2 = [jnp.where(i1 == e, neg_inf, v[e]) for e in range(NUM_EXPERTS)]
        m2 = l2[0]
        for e in range(1, NUM_EXPERTS):
            m2 = jnp.maximum(m2, l2[e])
        i2 = jnp.full((SC_LANES,), NUM_EXPERTS, jnp.int32)
        for e in reversed(range(NUM_EXPERTS)):
            i2 = jnp.where(l2[e] == m2, jnp.full((SC_LANES,), e, jnp.int32), i2)
        ed = jnp.exp(m2 - m1)
        p1 = 1.0 / (1.0 + ed)
        p2 = ed / (1.0 + ed)
        zero = jnp.zeros((SC_LANES,), jnp.float32)
        one = jnp.full((SC_LANES,), 1.0, jnp.float32)
        for e in range(NUM_EXPERTS):
            sel1 = i1 == e
            sel2 = i2 == e
            w_e = jnp.where(sel1, p1, zero) + jnp.where(sel2, p2, zero)
            plsc.store_scatter(w_v, [row, jnp.full((SC_LANES,), e, jnp.int32)], w_e)
            cnt[e] = cnt[e] + jnp.where(sel1, one, zero) + jnp.where(sel2, one, zero)
        i1_v[pl.ds(j * SC_LANES, SC_LANES)] = i1
        i2_v[pl.ds(j * SC_LANES, SC_LANES)] = i2

    for e in range(NUM_EXPERTS):
        cnt_v[e, :] = cnt[e]
    pltpu.sync_copy(w_v, w_hbm.at[pl.ds(base, tok_per_w), :])
    pltpu.sync_copy(i1_v, idxt_hbm.at[0, pl.ds(base, tok_per_w)])
    pltpu.sync_copy(i2_v, idxt_hbm.at[1, pl.ds(base, tok_per_w)])
    pltpu.sync_copy(cnt_v, cnt_hbm.at[wid])


def _stage3_first_body(h_ref, w_ref, w2_ref, b2_ref, exp_ref, out_ref):
    _stage3_compute(h_ref, w_ref, w2_ref, b2_ref, exp_ref, out_ref)


def _stage3_body(h_ref, w_ref, w2_ref, b2_ref, exp_ref, prev_ref, out_ref):
    del prev_ref
    _stage3_compute(h_ref, w_ref, w2_ref, b2_ref, exp_ref, out_ref)


def _stage3_compute(h_ref, w_ref, w2_ref, b2_ref, exp_ref, out_ref):
    w = w_ref[...]
    w_exp = jnp.dot(w, exp_ref[...], preferred_element_type=jnp.float32)
    hw = (h_ref[...].astype(jnp.float32) * w_exp).astype(jnp.bfloat16)
    out = jnp.dot(hw, w2_ref[...], preferred_element_type=jnp.float32)
    out = out + jnp.dot(w, b2_ref[...], preferred_element_type=jnp.float32)
    out_ref[...] = out


@jax.jit
def kernel(x, task_id, task_emb, gate_W, gate_b, W1, b1, W2, b2):
    B, S, D = x.shape
    T = B * S
    NH = 2
    Bh = B // NH
    Th = T // NH
    nblk = Th // TOK_BLK
    blk_per_b = nblk // Bh
    tok_per_w = Th // SC_WORKERS

    x2d = x.reshape(T, D)
    w1a = W1.transpose(1, 0, 2).reshape(D, EF).astype(jnp.bfloat16)
    b1f = b1.reshape(1, EF)
    gwx = gate_W[:D, :]
    gwt = gate_W[D:, :]
    gbr = gate_b.reshape(1, NUM_EXPERTS)
    w2a = W2.reshape(EF, D).astype(jnp.bfloat16)
    expand = jnp.repeat(jnp.eye(NUM_EXPERTS, dtype=jnp.float32), D_FFN, axis=1)
    expand = expand.reshape(NUM_EXPERTS, EF)
    mesh = plsc.VectorSubcoreMesh(core_axis_name="c", subcore_axis_name="s")

    grid_h = (nblk,)
    te_all = None
    routed = []
    for half in range(NH):
        xh = x2d[half * Th:(half + 1) * Th]
        h, s1, s2, te = pl.pallas_call(
            _stage1_body,
            grid=grid_h,
            in_specs=[
                pl.BlockSpec((TOK_BLK, D), lambda i: (i, 0)),
                pl.BlockSpec((D, EF), lambda i: (0, 0)),
                pl.BlockSpec((1, EF), lambda i: (0, 0)),
                pl.BlockSpec((NUM_TASKS, D_TASK_EMBED), lambda i: (0, 0)),
                pl.BlockSpec((D_TASK_EMBED, NUM_EXPERTS), lambda i: (0, 0)),
                pl.BlockSpec((1, NUM_EXPERTS), lambda i: (0, 0)),
            ],
            out_specs=[
                pl.BlockSpec((TOK_BLK, EF), lambda i: (i, 0)),
                pl.BlockSpec((1, 1, 128), lambda i: (i, 0, 0)),
                pl.BlockSpec((1, 1, 128), lambda i: (i, 0, 0)),
                pl.BlockSpec((NUM_TASKS, NUM_EXPERTS), lambda i: (0, 0)),
            ],
            out_shape=[
                jax.ShapeDtypeStruct((Th, EF), jnp.bfloat16),
                jax.ShapeDtypeStruct((nblk, 1, 128), jnp.float32),
                jax.ShapeDtypeStruct((nblk, 1, 128), jnp.float32),
                jax.ShapeDtypeStruct((NUM_TASKS, NUM_EXPERTS), jnp.float32),
            ],
        )(xh, w1a, b1f, task_emb, gwt, gbr)
        if te_all is None:
            te_all = te

        n = jnp.float32(S * D)
        bsum = s1[:, 0, 0].reshape(Bh, blk_per_b).sum(axis=1)
        bsq = s2[:, 0, 0].reshape(Bh, blk_per_b).sum(axis=1)
        mean = bsum / n
        var = bsq / n - mean * mean
        std = jnp.sqrt(var + 1e-5)
        mean_mat = jnp.broadcast_to(mean[:, None], (Bh, NUM_EXPERTS))
        std_mat = jnp.broadcast_to(std[:, None], (Bh, NUM_EXPERTS))
        te_row = te_all[task_id[half * Bh:(half + 1) * Bh]]

        logits = pl.pallas_call(
            _stage2_body,
            grid=grid_h,
            in_specs=[
                pl.BlockSpec((TOK_BLK, D), lambda i: (i, 0)),
                pl.BlockSpec((Bh, NUM_EXPERTS), lambda i: (0, 0)),
                pl.BlockSpec((Bh, NUM_EXPERTS), lambda i: (0, 0)),
                pl.BlockSpec((Bh, NUM_EXPERTS), lambda i: (0, 0)),
                pl.BlockSpec((D, NUM_EXPERTS), lambda i: (0, 0)),
            ],
            out_specs=pl.BlockSpec((TOK_BLK, NUM_EXPERTS), lambda i: (i, 0)),
            out_shape=jax.ShapeDtypeStruct((Th, NUM_EXPERTS), jnp.float32),
        )(xh, mean_mat, std_mat, te_row, gwx)

        w_tok, idxt, cnt = pl.kernel(
            _route_body,
            out_type=[
                jax.ShapeDtypeStruct((Th, NUM_EXPERTS), jnp.float32),
                jax.ShapeDtypeStruct((TOP_K, Th), jnp.int32),
                jax.ShapeDtypeStruct((SC_WORKERS, NUM_EXPERTS, SC_LANES),
                                     jnp.float32),
            ],
            mesh=mesh,
            compiler_params=pltpu.CompilerParams(needs_layout_passes=False),
            scratch_types=[
                pltpu.VMEM((tok_per_w, NUM_EXPERTS), jnp.float32),
                pltpu.VMEM((tok_per_w, NUM_EXPERTS), jnp.float32),
                pltpu.VMEM((tok_per_w,), jnp.int32),
                pltpu.VMEM((tok_per_w,), jnp.int32),
                pltpu.VMEM((NUM_EXPERTS, SC_LANES), jnp.float32),
            ],
        )(logits)
        routed.append((h, w_tok, idxt, cnt))

    out = None
    for half in range(NH):
        h, w_tok, _, _ = routed[half]
        off = half * nblk
        common_in_specs = [
            pl.BlockSpec((TOK_BLK, EF), lambda i: (i, 0)),
            pl.BlockSpec((TOK_BLK, NUM_EXPERTS), lambda i: (i, 0)),
            pl.BlockSpec((EF, D), lambda i: (0, 0)),
            pl.BlockSpec((NUM_EXPERTS, D), lambda i: (0, 0)),
            pl.BlockSpec((NUM_EXPERTS, EF), lambda i: (0, 0)),
        ]
        if half == 0:
            out = pl.pallas_call(
                _stage3_first_body,
                grid=grid_h,
                in_specs=common_in_specs,
                out_specs=pl.BlockSpec((TOK_BLK, D),
                                       lambda i, off=off: (i + off, 0)),
                out_shape=jax.ShapeDtypeStruct((T, D), jnp.float32),
            )(h, w_tok, w2a, b2, expand)
        else:
            out = pl.pallas_call(
                _stage3_body,
                grid=grid_h,
                in_specs=common_in_specs + [pl.BlockSpec(memory_space=pl.ANY)],
                out_specs=pl.BlockSpec((TOK_BLK, D),
                                       lambda i, off=off: (i + off, 0)),
                out_shape=jax.ShapeDtypeStruct((T, D), jnp.float32),
                input_output_aliases={5: 0},
            )(h, w_tok, w2a, b2, expand, out)

    final_output = out.reshape(B, S, D)
    idxt_full = jnp.concatenate([routed[0][2], routed[1][2]], axis=1)
    topk_idx = jnp.transpose(idxt_full, (1, 0)).reshape(B, S, TOP_K)

    counts = jnp.sum(routed[0][3], axis=(0, 2)) + jnp.sum(routed[1][3], axis=(0, 2))
    usage_mean = jnp.mean(counts) + 1e-6
    usage_std = jnp.std(counts, ddof=1)
    lb_loss = (usage_std / usage_mean) ** 2
    return (final_output, lb_loss, topk_idx)


# speedup vs baseline: 1.0091x; 1.0091x over previous
import jax
import jax.numpy as jnp
from jax import lax
from jax.experimental import pallas as pl
from jax.experimental.pallas import tpu as pltpu
from jax.experimental.pallas import tpu_sc as plsc

D_MODEL = 2048
NUM_EXPERTS = 8
TOP_K = 2
NUM_TASKS = 64
D_TASK_EMBED = 64
D_FFN = 32
EF = NUM_EXPERTS * D_FFN

TOK_BLK = 512

SC_CORES = 2
SC_SUBCORES = 16
SC_WORKERS = SC_CORES * SC_SUBCORES
SC_LANES = 16


def _stage1_body(x_ref, w1_ref, b1_ref, temb_ref, gwt_ref, gb_ref,
                 h_ref, s1_ref, s2_ref, te_ref):
    x = x_ref[...]
    s1_ref[...] = jnp.full((1, 1, 128), jnp.sum(x), dtype=jnp.float32)
    s2_ref[...] = jnp.full((1, 1, 128), jnp.sum(x * x), dtype=jnp.float32)
    h = jnp.dot(x.astype(jnp.bfloat16), w1_ref[...],
                preferred_element_type=jnp.float32)
    h = h + b1_ref[...]
    h_ref[...] = (h * jax.nn.sigmoid(h)).astype(jnp.bfloat16)
    te_ref[...] = jnp.dot(temb_ref[...].astype(jnp.bfloat16),
                          gwt_ref[...].astype(jnp.bfloat16),
                          preferred_element_type=jnp.float32) + gb_ref[...]


def _stage2_body(x_ref, mean_ref, std_ref, te_ref, gwx_ref, lg_ref):
    i = pl.program_id(0)
    b = i // (pl.num_programs(0) // mean_ref.shape[0])
    mean_row = mean_ref[pl.ds(b, 1), :][:, :1]
    std_row = std_ref[pl.ds(b, 1), :][:, :1]
    te_row = te_ref[pl.ds(b, 1), :]
    xn = (x_ref[...] - mean_row) / std_row
    lg_ref[...] = jnp.dot(xn.astype(jnp.bfloat16),
                          gwx_ref[...].astype(jnp.bfloat16),
                          preferred_element_type=jnp.float32) + te_row


def _route_body(lg_hbm, w_hbm, idxt_hbm, cnt_hbm, lg_v, w_v, i1_v, i2_v, cnt_v):
    tok_per_w = lg_hbm.shape[0] // SC_WORKERS
    n_grp = tok_per_w // SC_LANES
    wid = lax.axis_index("s") * SC_CORES + lax.axis_index("c")
    base = wid * tok_per_w
    pltpu.sync_copy(lg_hbm.at[pl.ds(base, tok_per_w), :], lg_v)

    lane = lax.iota(jnp.int32, SC_LANES)
    neg_inf = jnp.full((SC_LANES,), -jnp.inf, jnp.float32)
    cnt = [jnp.zeros((SC_LANES,), jnp.float32) for _ in range(NUM_EXPERTS)]

    for j in range(n_grp):
        row = jnp.full((SC_LANES,), j * SC_LANES, jnp.int32) + lane
        v = [plsc.load_gather(lg_v, [row, jnp.full((SC_LANES,), e, jnp.int32)])
             for e in range(NUM_EXPERTS)]
        m1 = v[0]
        for e in range(1, NUM_EXPERTS):
            m1 = jnp.maximum(m1, v[e])
        i1 = jnp.full((SC_LANES,), NUM_EXPERTS, jnp.int32)
        for e in reversed(range(NUM_EXPERTS)):
            i1 = jnp.where(v[e] == m1, jnp.full((SC_LANES,), e, jnp.int32), i1)
        l2 = [jnp.where(i1 == e, neg_inf, v[e]) for e in range(NUM_EXPERTS)]
        m2 = l2[0]
        for e in range(1, NUM_EXPERTS):
            m2 = jnp.maximum(m2, l2[e])
        i2 = jnp.full((SC_LANES,), NUM_EXPERTS, jnp.int32)
        for e in reversed(range(NUM_EXPERTS)):
            i2 = jnp.where(l2[e] == m2, jnp.full((SC_LANES,), e, jnp.int32), i2)
        ed = jnp.exp(m2 - m1)
        p1 = 1.0 / (1.0 + ed)
        p2 = ed / (1.0 + ed)
        zero = jnp.zeros((SC_LANES,), jnp.float32)
        one = jnp.full((SC_LANES,), 1.0, jnp.float32)
        for e in range(NUM_EXPERTS):
            sel1 = i1 == e
            sel2 = i2 == e
            w_e = jnp.where(sel1, p1, zero) + jnp.where(sel2, p2, zero)
            plsc.store_scatter(w_v, [row, jnp.full((SC_LANES,), e, jnp.int32)], w_e)
            cnt[e] = cnt[e] + jnp.where(sel1, one, zero) + jnp.where(sel2, one, zero)
        i1_v[pl.ds(j * SC_LANES, SC_LANES)] = i1
        i2_v[pl.ds(j * SC_LANES, SC_LANES)] = i2

    for e in range(NUM_EXPERTS):
        cnt_v[e, :] = cnt[e]
    pltpu.sync_copy(w_v, w_hbm.at[pl.ds(base, tok_per_w), :])
    pltpu.sync_copy(i1_v, idxt_hbm.at[0, pl.ds(base, tok_per_w)])
    pltpu.sync_copy(i2_v, idxt_hbm.at[1, pl.ds(base, tok_per_w)])
    pltpu.sync_copy(cnt_v, cnt_hbm.at[wid])


def _stage3_body(h0_ref, h1_ref, w0_ref, w1_ref, w2_ref, b2_ref, exp_ref,
                 out_ref):
    nh = pl.num_programs(0) // 2
    in_first = pl.program_id(0) < nh
    h = jnp.where(in_first, h0_ref[...].astype(jnp.float32),
                  h1_ref[...].astype(jnp.float32))
    w = jnp.where(in_first, w0_ref[...], w1_ref[...])
    w_exp = jnp.dot(w, exp_ref[...], preferred_element_type=jnp.float32)
    hw = (h * w_exp).astype(jnp.bfloat16)
    out = jnp.dot(hw, w2_ref[...], preferred_element_type=jnp.float32)
    out = out + jnp.dot(w, b2_ref[...], preferred_element_type=jnp.float32)
    out_ref[...] = out


@jax.jit
def kernel(x, task_id, task_emb, gate_W, gate_b, W1, b1, W2, b2):
    B, S, D = x.shape
    T = B * S
    NH = 2
    Bh = B // NH
    Th = T // NH
    nblk = Th // TOK_BLK
    blk_per_b = nblk // Bh
    tok_per_w = Th // SC_WORKERS

    x2d = x.reshape(T, D)
    w1a = W1.transpose(1, 0, 2).reshape(D, EF).astype(jnp.bfloat16)
    b1f = b1.reshape(1, EF)
    gwx = gate_W[:D, :]
    gwt = gate_W[D:, :]
    gbr = gate_b.reshape(1, NUM_EXPERTS)
    w2a = W2.reshape(EF, D).astype(jnp.bfloat16)
    expand = jnp.repeat(jnp.eye(NUM_EXPERTS, dtype=jnp.float32), D_FFN, axis=1)
    expand = expand.reshape(NUM_EXPERTS, EF)
    mesh = plsc.VectorSubcoreMesh(core_axis_name="c", subcore_axis_name="s")

    grid_h = (nblk,)
    te_all = None
    routed = []
    for half in range(NH):
        xh = x2d[half * Th:(half + 1) * Th]
        h, s1, s2, te = pl.pallas_call(
            _stage1_body,
            grid=grid_h,
            in_specs=[
                pl.BlockSpec((TOK_BLK, D), lambda i: (i, 0)),
                pl.BlockSpec((D, EF), lambda i: (0, 0)),
                pl.BlockSpec((1, EF), lambda i: (0, 0)),
                pl.BlockSpec((NUM_TASKS, D_TASK_EMBED), lambda i: (0, 0)),
                pl.BlockSpec((D_TASK_EMBED, NUM_EXPERTS), lambda i: (0, 0)),
                pl.BlockSpec((1, NUM_EXPERTS), lambda i: (0, 0)),
            ],
            out_specs=[
                pl.BlockSpec((TOK_BLK, EF), lambda i: (i, 0)),
                pl.BlockSpec((1, 1, 128), lambda i: (i, 0, 0)),
                pl.BlockSpec((1, 1, 128), lambda i: (i, 0, 0)),
                pl.BlockSpec((NUM_TASKS, NUM_EXPERTS), lambda i: (0, 0)),
            ],
            out_shape=[
                jax.ShapeDtypeStruct((Th, EF), jnp.bfloat16),
                jax.ShapeDtypeStruct((nblk, 1, 128), jnp.float32),
                jax.ShapeDtypeStruct((nblk, 1, 128), jnp.float32),
                jax.ShapeDtypeStruct((NUM_TASKS, NUM_EXPERTS), jnp.float32),
            ],
        )(xh, w1a, b1f, task_emb, gwt, gbr)
        if te_all is None:
            te_all = te

        n = jnp.float32(S * D)
        bsum = s1[:, 0, 0].reshape(Bh, blk_per_b).sum(axis=1)
        bsq = s2[:, 0, 0].reshape(Bh, blk_per_b).sum(axis=1)
        mean = bsum / n
        var = bsq / n - mean * mean
        std = jnp.sqrt(var + 1e-5)
        mean_mat = jnp.broadcast_to(mean[:, None], (Bh, NUM_EXPERTS))
        std_mat = jnp.broadcast_to(std[:, None], (Bh, NUM_EXPERTS))
        te_row = te_all[task_id[half * Bh:(half + 1) * Bh]]

        logits = pl.pallas_call(
            _stage2_body,
            grid=grid_h,
            in_specs=[
                pl.BlockSpec((TOK_BLK, D), lambda i: (i, 0)),
                pl.BlockSpec((Bh, NUM_EXPERTS), lambda i: (0, 0)),
                pl.BlockSpec((Bh, NUM_EXPERTS), lambda i: (0, 0)),
                pl.BlockSpec((Bh, NUM_EXPERTS), lambda i: (0, 0)),
                pl.BlockSpec((D, NUM_EXPERTS), lambda i: (0, 0)),
            ],
            out_specs=pl.BlockSpec((TOK_BLK, NUM_EXPERTS), lambda i: (i, 0)),
            out_shape=jax.ShapeDtypeStruct((Th, NUM_EXPERTS), jnp.float32),
        )(xh, mean_mat, std_mat, te_row, gwx)

        w_tok, idxt, cnt = pl.kernel(
            _route_body,
            out_type=[
                jax.ShapeDtypeStruct((Th, NUM_EXPERTS), jnp.float32),
                jax.ShapeDtypeStruct((TOP_K, Th), jnp.int32),
                jax.ShapeDtypeStruct((SC_WORKERS, NUM_EXPERTS, SC_LANES),
                                     jnp.float32),
            ],
            mesh=mesh,
            compiler_params=pltpu.CompilerParams(needs_layout_passes=False),
            scratch_types=[
                pltpu.VMEM((tok_per_w, NUM_EXPERTS), jnp.float32),
                pltpu.VMEM((tok_per_w, NUM_EXPERTS), jnp.float32),
                pltpu.VMEM((tok_per_w,), jnp.int32),
                pltpu.VMEM((tok_per_w,), jnp.int32),
                pltpu.VMEM((NUM_EXPERTS, SC_LANES), jnp.float32),
            ],
        )(logits)
        routed.append((h, w_tok, idxt, cnt))

    nblk_f = nblk * NH
    clamp_lo = lambda i: (jnp.minimum(i, nblk - 1), 0)
    clamp_hi = lambda i: (jnp.maximum(i - nblk, 0), 0)
    out = pl.pallas_call(
        _stage3_body,
        grid=(nblk_f,),
        in_specs=[
            pl.BlockSpec((TOK_BLK, EF), clamp_lo),
            pl.BlockSpec((TOK_BLK, EF), clamp_hi),
            pl.BlockSpec((TOK_BLK, NUM_EXPERTS), clamp_lo),
            pl.BlockSpec((TOK_BLK, NUM_EXPERTS), clamp_hi),
            pl.BlockSpec((EF, D), lambda i: (0, 0)),
            pl.BlockSpec((NUM_EXPERTS, D), lambda i: (0, 0)),
            pl.BlockSpec((NUM_EXPERTS, EF), lambda i: (0, 0)),
        ],
        out_specs=pl.BlockSpec((TOK_BLK, D), lambda i: (i, 0)),
        out_shape=jax.ShapeDtypeStruct((T, D), jnp.float32),
    )(routed[0][0], routed[1][0], routed[0][1], routed[1][1], w2a, b2, expand)

    final_output = out.reshape(B, S, D)
    idxt_full = jnp.concatenate([routed[0][2], routed[1][2]], axis=1)
    topk_idx = jnp.transpose(idxt_full, (1, 0)).reshape(B, S, TOP_K)

    counts = jnp.sum(routed[0][3], axis=(0, 2)) + jnp.sum(routed[1][3], axis=(0, 2))
    usage_mean = jnp.mean(counts) + 1e-6
    usage_std = jnp.std(counts, ddof=1)
    lb_loss = (usage_std / usage_mean) ** 2
    return (final_output, lb_loss, topk_idx)


# speedup vs baseline: 1.4041x; 1.3915x over previous
import jax
import jax.numpy as jnp
from jax import lax
from jax.experimental import pallas as pl
from jax.experimental.pallas import tpu as pltpu
from jax.experimental.pallas import tpu_sc as plsc

D_MODEL = 2048
NUM_EXPERTS = 8
TOP_K = 2
NUM_TASKS = 64
D_TASK_EMBED = 64
D_FFN = 32
EF = NUM_EXPERTS * D_FFN

TOK_BLK = 512

SC_CORES = 2
SC_SUBCORES = 16
SC_WORKERS = SC_CORES * SC_SUBCORES
SC_LANES = 16


def _stage12_body(x_ref, w1_ref, b1_ref, gwx_ref, temb_ref, gwt_ref, gb_ref,
                  h_ref, lg_ref, stats_ref):
    i = pl.program_id(0)
    nb = pl.num_programs(0) // 2
    blk_per_b = nb // stats_ref.shape[1]
    n_elem = jnp.float32(blk_per_b * TOK_BLK * D_MODEL)
    x = x_ref[...]

    @pl.when(i == 0)
    def _init():
        for b in range(stats_ref.shape[1]):
            stats_ref[0, b] = 0.0
            stats_ref[1, b] = 0.0

    @pl.when(i < nb)
    def _phase_a():
        b = i // blk_per_b
        stats_ref[0, b] += jnp.sum(x)
        stats_ref[1, b] += jnp.sum(x * x)
        h = jnp.dot(x.astype(jnp.bfloat16), w1_ref[...],
                    preferred_element_type=jnp.float32)
        h = h + b1_ref[...]
        h_ref[...] = (h * jax.nn.sigmoid(h)).astype(jnp.bfloat16)

    @pl.when(i >= nb)
    def _phase_b():
        b = (i - nb) // blk_per_b
        mean = stats_ref[0, b] / n_elem
        var = stats_ref[1, b] / n_elem - mean * mean
        std = jnp.sqrt(var + 1e-5)
        xn = (x - mean) / std
        tb = temb_ref[pl.ds(b, 1), :]
        te_row = jnp.dot(tb.astype(jnp.bfloat16),
                         gwt_ref[...].astype(jnp.bfloat16),
                         preferred_element_type=jnp.float32) + gb_ref[...]
        lg_ref[...] = jnp.dot(xn.astype(jnp.bfloat16),
                              gwx_ref[...].astype(jnp.bfloat16),
                              preferred_element_type=jnp.float32) + te_row


def _route_body(lg_hbm, w_hbm, idxt_hbm, cnt_hbm, lg_v, w_v, i1_v, i2_v, cnt_v):
    tok_per_w = lg_hbm.shape[0] // SC_WORKERS
    n_grp = tok_per_w // SC_LANES
    wid = lax.axis_index("s") * SC_CORES + lax.axis_index("c")
    base = wid * tok_per_w
    pltpu.sync_copy(lg_hbm.at[pl.ds(base, tok_per_w), :], lg_v)

    lane = lax.iota(jnp.int32, SC_LANES)
    neg_inf = jnp.full((SC_LANES,), -jnp.inf, jnp.float32)
    cnt = [jnp.zeros((SC_LANES,), jnp.float32) for _ in range(NUM_EXPERTS)]

    for j in range(n_grp):
        row = jnp.full((SC_LANES,), j * SC_LANES, jnp.int32) + lane
        v = [plsc.load_gather(lg_v, [row, jnp.full((SC_LANES,), e, jnp.int32)])
             for e in range(NUM_EXPERTS)]
        m1 = v[0]
        for e in range(1, NUM_EXPERTS):
            m1 = jnp.maximum(m1, v[e])
        i1 = jnp.full((SC_LANES,), NUM_EXPERTS, jnp.int32)
        for e in reversed(range(NUM_EXPERTS)):
            i1 = jnp.where(v[e] == m1, jnp.full((SC_LANES,), e, jnp.int32), i1)
        l2 = [jnp.where(i1 == e, neg_inf, v[e]) for e in range(NUM_EXPERTS)]
        m2 = l2[0]
        for e in range(1, NUM_EXPERTS):
            m2 = jnp.maximum(m2, l2[e])
        i2 = jnp.full((SC_LANES,), NUM_EXPERTS, jnp.int32)
        for e in reversed(range(NUM_EXPERTS)):
            i2 = jnp.where(l2[e] == m2, jnp.full((SC_LANES,), e, jnp.int32), i2)
        ed = jnp.exp(m2 - m1)
        p1 = 1.0 / (1.0 + ed)
        p2 = ed / (1.0 + ed)
        zero = jnp.zeros((SC_LANES,), jnp.float32)
        one = jnp.full((SC_LANES,), 1.0, jnp.float32)
        for e in range(NUM_EXPERTS):
            sel1 = i1 == e
            sel2 = i2 == e
            w_e = jnp.where(sel1, p1, zero) + jnp.where(sel2, p2, zero)
            plsc.store_scatter(w_v, [row, jnp.full((SC_LANES,), e, jnp.int32)], w_e)
            cnt[e] = cnt[e] + jnp.where(sel1, one, zero) + jnp.where(sel2, one, zero)
        i1_v[pl.ds(j * SC_LANES, SC_LANES)] = i1
        i2_v[pl.ds(j * SC_LANES, SC_LANES)] = i2

    for e in range(NUM_EXPERTS):
        cnt_v[e, :] = cnt[e]
    pltpu.sync_copy(w_v, w_hbm.at[pl.ds(base, tok_per_w), :])
    pltpu.sync_copy(i1_v, idxt_hbm.at[0, pl.ds(base, tok_per_w)])
    pltpu.sync_copy(i2_v, idxt_hbm.at[1, pl.ds(base, tok_per_w)])
    pltpu.sync_copy(cnt_v, cnt_hbm.at[wid])


def _stage3_body(h_ref, w_ref, w2_ref, b2_ref, exp_ref, out_ref):
    w = w_ref[...]
    w_exp = jnp.dot(w, exp_ref[...], preferred_element_type=jnp.float32)
    hw = (h_ref[...].astype(jnp.float32) * w_exp).astype(jnp.bfloat16)
    out = jnp.dot(hw, w2_ref[...], preferred_element_type=jnp.float32)
    out = out + jnp.dot(w, b2_ref[...], preferred_element_type=jnp.float32)
    out_ref[...] = out


@jax.jit
def kernel(x, task_id, task_emb, gate_W, gate_b, W1, b1, W2, b2):
    B, S, D = x.shape
    T = B * S
    nblk = T // TOK_BLK
    tok_per_w = T // SC_WORKERS

    x2d = x.reshape(T, D)
    w1a = W1.transpose(1, 0, 2).reshape(D, EF).astype(jnp.bfloat16)
    b1f = b1.reshape(1, EF)
    gwx = gate_W[:D, :]
    gwt = gate_W[D:, :]
    gbr = gate_b.reshape(1, NUM_EXPERTS)
    w2a = W2.reshape(EF, D).astype(jnp.bfloat16)
    temb_sel = task_emb[task_id]

    h, logits = pl.pallas_call(
        _stage12_body,
        grid=(2 * nblk,),
        in_specs=[
            pl.BlockSpec((TOK_BLK, D),
                         lambda i: (jnp.where(i < nblk, i, i - nblk), 0)),
            pl.BlockSpec((D, EF), lambda i: (0, 0)),
            pl.BlockSpec((1, EF), lambda i: (0, 0)),
            pl.BlockSpec((D, NUM_EXPERTS), lambda i: (0, 0)),
            pl.BlockSpec((B, D_TASK_EMBED), lambda i: (0, 0)),
            pl.BlockSpec((D_TASK_EMBED, NUM_EXPERTS), lambda i: (0, 0)),
            pl.BlockSpec((1, NUM_EXPERTS), lambda i: (0, 0)),
        ],
        out_specs=[
            pl.BlockSpec((TOK_BLK, EF), lambda i: (jnp.minimum(i, nblk - 1), 0)),
            pl.BlockSpec((TOK_BLK, NUM_EXPERTS),
                         lambda i: (jnp.maximum(i - nblk, 0), 0)),
        ],
        out_shape=[
            jax.ShapeDtypeStruct((T, EF), jnp.bfloat16),
            jax.ShapeDtypeStruct((T, NUM_EXPERTS), jnp.float32),
        ],
        scratch_shapes=[pltpu.SMEM((2, B), jnp.float32)],
    )(x2d, w1a, b1f, gwx, temb_sel, gwt, gbr)

    mesh = plsc.VectorSubcoreMesh(core_axis_name="c", subcore_axis_name="s")
    w_tok, idxt, cnt = pl.kernel(
        _route_body,
        out_type=[
            jax.ShapeDtypeStruct((T, NUM_EXPERTS), jnp.float32),
            jax.ShapeDtypeStruct((TOP_K, T), jnp.int32),
            jax.ShapeDtypeStruct((SC_WORKERS, NUM_EXPERTS, SC_LANES),
                                 jnp.float32),
        ],
        mesh=mesh,
        compiler_params=pltpu.CompilerParams(needs_layout_passes=False),
        scratch_types=[
            pltpu.VMEM((tok_per_w, NUM_EXPERTS), jnp.float32),
            pltpu.VMEM((tok_per_w, NUM_EXPERTS), jnp.float32),
            pltpu.VMEM((tok_per_w,), jnp.int32),
            pltpu.VMEM((tok_per_w,), jnp.int32),
            pltpu.VMEM((NUM_EXPERTS, SC_LANES), jnp.float32),
        ],
    )(logits)

    expand = jnp.repeat(jnp.eye(NUM_EXPERTS, dtype=jnp.float32), D_FFN, axis=1)
    expand = expand.reshape(NUM_EXPERTS, EF)

    out = pl.pallas_call(
        _stage3_body,
        grid=(nblk,),
        in_specs=[
            pl.BlockSpec((TOK_BLK, EF), lambda i: (i, 0)),
            pl.BlockSpec((TOK_BLK, NUM_EXPERTS), lambda i: (i, 0)),
            pl.BlockSpec((EF, D), lambda i: (0, 0)),
            pl.BlockSpec((NUM_EXPERTS, D), lambda i: (0, 0)),
            pl.BlockSpec((NUM_EXPERTS, EF), lambda i: (0, 0)),
        ],
        out_specs=pl.BlockSpec((TOK_BLK, D), lambda i: (i, 0)),
        out_shape=jax.ShapeDtypeStruct((T, D), jnp.float32),
    )(h, w_tok, w2a, b2, expand)

    final_output = out.reshape(B, S, D)
    topk_idx = jnp.transpose(idxt, (1, 0)).reshape(B, S, TOP_K)

    counts = jnp.sum(cnt, axis=(0, 2))
    usage_mean = jnp.mean(counts) + 1e-6
    usage_std = jnp.std(counts, ddof=1)
    lb_loss = (usage_std / usage_mean) ** 2
    return (final_output, lb_loss, topk_idx)


# speedup vs baseline: 1.4191x; 1.0107x over previous
import jax
import jax.numpy as jnp
from jax import lax
from jax.experimental import pallas as pl
from jax.experimental.pallas import tpu as pltpu
from jax.experimental.pallas import tpu_sc as plsc

D_MODEL = 2048
NUM_EXPERTS = 8
TOP_K = 2
NUM_TASKS = 64
D_TASK_EMBED = 64
D_FFN = 32
EF = NUM_EXPERTS * D_FFN

TOK_BLK = 512

SC_CORES = 2
SC_SUBCORES = 16
SC_WORKERS = SC_CORES * SC_SUBCORES
SC_LANES = 16


def _stage12_body(x_ref, w1_ref, b1_ref, gwx_ref, temb_ref, gwt_ref, gb_ref,
                  h_ref, lg_ref, w_ref, stats_ref):
    i = pl.program_id(0)
    nb = pl.num_programs(0) // 2
    blk_per_b = nb // stats_ref.shape[1]
    n_elem = jnp.float32(blk_per_b * TOK_BLK * D_MODEL)
    x = x_ref[...]

    @pl.when(i == 0)
    def _init():
        for b in range(stats_ref.shape[1]):
            stats_ref[0, b] = 0.0
            stats_ref[1, b] = 0.0

    @pl.when(i < nb)
    def _phase_a():
        b = i // blk_per_b
        stats_ref[0, b] += jnp.sum(x)
        stats_ref[1, b] += jnp.sum(x * x)
        h = jnp.dot(x.astype(jnp.bfloat16), w1_ref[...],
                    preferred_element_type=jnp.float32)
        h = h + b1_ref[...]
        h_ref[...] = (h * jax.nn.sigmoid(h)).astype(jnp.bfloat16)

    @pl.when(i >= nb)
    def _phase_b():
        b = (i - nb) // blk_per_b
        mean = stats_ref[0, b] / n_elem
        var = stats_ref[1, b] / n_elem - mean * mean
        std = jnp.sqrt(var + 1e-5)
        xn = (x - mean) / std
        tb = temb_ref[pl.ds(b, 1), :]
        te_row = jnp.dot(tb.astype(jnp.bfloat16),
                         gwt_ref[...].astype(jnp.bfloat16),
                         preferred_element_type=jnp.float32) + gb_ref[...]
        logits = jnp.dot(xn.astype(jnp.bfloat16),
                         gwx_ref[...].astype(jnp.bfloat16),
                         preferred_element_type=jnp.float32) + te_row
        lg_ref[...] = logits
        ii = lax.broadcasted_iota(jnp.int32, logits.shape, 1)
        m1 = jnp.max(logits, axis=1, keepdims=True)
        i1 = jnp.min(jnp.where(logits == m1, ii, NUM_EXPERTS), axis=1,
                     keepdims=True)
        l2 = jnp.where(ii == i1, -jnp.inf, logits)
        m2 = jnp.max(l2, axis=1, keepdims=True)
        i2 = jnp.min(jnp.where(l2 == m2, ii, NUM_EXPERTS), axis=1,
                     keepdims=True)
        ed = jnp.exp(m2 - m1)
        p1 = 1.0 / (1.0 + ed)
        p2 = ed / (1.0 + ed)
        w_ref[...] = jnp.where(ii == i1, p1, 0.0) + jnp.where(ii == i2, p2, 0.0)


def _route_body(lg_hbm, idxt_hbm, cnt_hbm, lg_v, i1_v, i2_v, cnt_v):
    tok_per_w = lg_hbm.shape[0] // SC_WORKERS
    n_grp = tok_per_w // SC_LANES
    wid = lax.axis_index("s") * SC_CORES + lax.axis_index("c")
    base = wid * tok_per_w
    pltpu.sync_copy(lg_hbm.at[pl.ds(base, tok_per_w), :], lg_v)

    lane = lax.iota(jnp.int32, SC_LANES)
    neg_inf = jnp.full((SC_LANES,), -jnp.inf, jnp.float32)
    cnt = [jnp.zeros((SC_LANES,), jnp.float32) for _ in range(NUM_EXPERTS)]

    for j in range(n_grp):
        row = jnp.full((SC_LANES,), j * SC_LANES, jnp.int32) + lane
        v = [plsc.load_gather(lg_v, [row, jnp.full((SC_LANES,), e, jnp.int32)])
             for e in range(NUM_EXPERTS)]
        m1 = v[0]
        for e in range(1, NUM_EXPERTS):
            m1 = jnp.maximum(m1, v[e])
        i1 = jnp.full((SC_LANES,), NUM_EXPERTS, jnp.int32)
        for e in reversed(range(NUM_EXPERTS)):
            i1 = jnp.where(v[e] == m1, jnp.full((SC_LANES,), e, jnp.int32), i1)
        l2 = [jnp.where(i1 == e, neg_inf, v[e]) for e in range(NUM_EXPERTS)]
        m2 = l2[0]
        for e in range(1, NUM_EXPERTS):
            m2 = jnp.maximum(m2, l2[e])
        i2 = jnp.full((SC_LANES,), NUM_EXPERTS, jnp.int32)
        for e in reversed(range(NUM_EXPERTS)):
            i2 = jnp.where(l2[e] == m2, jnp.full((SC_LANES,), e, jnp.int32), i2)
        zero = jnp.zeros((SC_LANES,), jnp.float32)
        one = jnp.full((SC_LANES,), 1.0, jnp.float32)
        for e in range(NUM_EXPERTS):
            cnt[e] = (cnt[e] + jnp.where(i1 == e, one, zero)
                      + jnp.where(i2 == e, one, zero))
        i1_v[pl.ds(j * SC_LANES, SC_LANES)] = i1
        i2_v[pl.ds(j * SC_LANES, SC_LANES)] = i2

    for e in range(NUM_EXPERTS):
        cnt_v[e, :] = cnt[e]
    pltpu.sync_copy(i1_v, idxt_hbm.at[0, pl.ds(base, tok_per_w)])
    pltpu.sync_copy(i2_v, idxt_hbm.at[1, pl.ds(base, tok_per_w)])
    pltpu.sync_copy(cnt_v, cnt_hbm.at[wid])


def _stage3_body(h_ref, w_ref, w2_ref, b2_ref, exp_ref, out_ref):
    w = w_ref[...]
    w_exp = jnp.dot(w, exp_ref[...], preferred_element_type=jnp.float32)
    hw = (h_ref[...].astype(jnp.float32) * w_exp).astype(jnp.bfloat16)
    out = jnp.dot(hw, w2_ref[...], preferred_element_type=jnp.float32)
    out = out + jnp.dot(w, b2_ref[...], preferred_element_type=jnp.float32)
    out_ref[...] = out


@jax.jit
def kernel(x, task_id, task_emb, gate_W, gate_b, W1, b1, W2, b2):
    B, S, D = x.shape
    T = B * S
    nblk = T // TOK_BLK
    tok_per_w = T // SC_WORKERS

    x2d = x.reshape(T, D)
    w1a = W1.transpose(1, 0, 2).reshape(D, EF).astype(jnp.bfloat16)
    b1f = b1.reshape(1, EF)
    gwx = gate_W[:D, :]
    gwt = gate_W[D:, :]
    gbr = gate_b.reshape(1, NUM_EXPERTS)
    w2a = W2.reshape(EF, D).astype(jnp.bfloat16)
    temb_sel = task_emb[task_id]

    h, logits, w_tok = pl.pallas_call(
        _stage12_body,
        grid=(2 * nblk,),
        in_specs=[
            pl.BlockSpec((TOK_BLK, D),
                         lambda i: (jnp.where(i < nblk, i, i - nblk), 0)),
            pl.BlockSpec((D, EF), lambda i: (0, 0)),
            pl.BlockSpec((1, EF), lambda i: (0, 0)),
            pl.BlockSpec((D, NUM_EXPERTS), lambda i: (0, 0)),
            pl.BlockSpec((B, D_TASK_EMBED), lambda i: (0, 0)),
            pl.BlockSpec((D_TASK_EMBED, NUM_EXPERTS), lambda i: (0, 0)),
            pl.BlockSpec((1, NUM_EXPERTS), lambda i: (0, 0)),
        ],
        out_specs=[
            pl.BlockSpec((TOK_BLK, EF), lambda i: (jnp.minimum(i, nblk - 1), 0)),
            pl.BlockSpec((TOK_BLK, NUM_EXPERTS),
                         lambda i: (jnp.maximum(i - nblk, 0), 0)),
            pl.BlockSpec((TOK_BLK, NUM_EXPERTS),
                         lambda i: (jnp.maximum(i - nblk, 0), 0)),
        ],
        out_shape=[
            jax.ShapeDtypeStruct((T, EF), jnp.bfloat16),
            jax.ShapeDtypeStruct((T, NUM_EXPERTS), jnp.float32),
            jax.ShapeDtypeStruct((T, NUM_EXPERTS), jnp.float32),
        ],
        scratch_shapes=[pltpu.SMEM((2, B), jnp.float32)],
    )(x2d, w1a, b1f, gwx, temb_sel, gwt, gbr)

    mesh = plsc.VectorSubcoreMesh(core_axis_name="c", subcore_axis_name="s")
    idxt, cnt = pl.kernel(
        _route_body,
        out_type=[
            jax.ShapeDtypeStruct((TOP_K, T), jnp.int32),
            jax.ShapeDtypeStruct((SC_WORKERS, NUM_EXPERTS, SC_LANES),
                                 jnp.float32),
        ],
        mesh=mesh,
        compiler_params=pltpu.CompilerParams(needs_layout_passes=False),
        scratch_types=[
            pltpu.VMEM((tok_per_w, NUM_EXPERTS), jnp.float32),
            pltpu.VMEM((tok_per_w,), jnp.int32),
            pltpu.VMEM((tok_per_w,), jnp.int32),
            pltpu.VMEM((NUM_EXPERTS, SC_LANES), jnp.float32),
        ],
    )(logits)

    expand = jnp.repeat(jnp.eye(NUM_EXPERTS, dtype=jnp.float32), D_FFN, axis=1)
    expand = expand.reshape(NUM_EXPERTS, EF)

    out = pl.pallas_call(
        _stage3_body,
        grid=(nblk,),
        in_specs=[
            pl.BlockSpec((TOK_BLK, EF), lambda i: (i, 0)),
            pl.BlockSpec((TOK_BLK, NUM_EXPERTS), lambda i: (i, 0)),
            pl.BlockSpec((EF, D), lambda i: (0, 0)),
            pl.BlockSpec((NUM_EXPERTS, D), lambda i: (0, 0)),
            pl.BlockSpec((NUM_EXPERTS, EF), lambda i: (0, 0)),
        ],
        out_specs=pl.BlockSpec((TOK_BLK, D), lambda i: (i, 0)),
        out_shape=jax.ShapeDtypeStruct((T, D), jnp.float32),
    )(h, w_tok, w2a, b2, expand)

    final_output = out.reshape(B, S, D)
    topk_idx = jnp.transpose(idxt, (1, 0)).reshape(B, S, TOP_K)

    counts = jnp.sum(cnt, axis=(0, 2))
    usage_mean = jnp.mean(counts) + 1e-6
    usage_std = jnp.std(counts, ddof=1)
    lb_loss = (usage_std / usage_mean) ** 2
    return (final_output, lb_loss, topk_idx)


# speedup vs baseline: 1.4553x; 1.0255x over previous
import jax
import jax.numpy as jnp
from jax import lax
from jax.experimental import pallas as pl
from jax.experimental.pallas import tpu as pltpu
from jax.experimental.pallas import tpu_sc as plsc

D_MODEL = 2048
NUM_EXPERTS = 8
TOP_K = 2
NUM_TASKS = 64
D_TASK_EMBED = 64
D_FFN = 32
EF = NUM_EXPERTS * D_FFN

TOK_BLK = 512

SC_CORES = 2
SC_SUBCORES = 16
SC_WORKERS = SC_CORES * SC_SUBCORES
SC_LANES = 16


def _stage12_body(tid_ref, x_ref, w1_ref, b1_ref, gw_ref, gb_ref, temb_ref,
                  h_ref, lg_ref, w_ref, stats_ref):
    i = pl.program_id(0)
    nb = pl.num_programs(0) // 2
    blk_per_b = nb // stats_ref.shape[1]
    n_elem = jnp.float32(blk_per_b * TOK_BLK * D_MODEL)
    x = x_ref[...]

    @pl.when(i == 0)
    def _init():
        for b in range(stats_ref.shape[1]):
            stats_ref[0, b] = 0.0
            stats_ref[1, b] = 0.0

    @pl.when(i < nb)
    def _phase_a():
        b = i // blk_per_b
        stats_ref[0, b] += jnp.sum(x)
        stats_ref[1, b] += jnp.sum(x * x)
        h = jnp.dot(x.astype(jnp.bfloat16), w1_ref[...],
                    preferred_element_type=jnp.float32)
        h = h + b1_ref[...]
        h_ref[...] = (h * jax.nn.sigmoid(h)).astype(jnp.bfloat16)

    @pl.when(i >= nb)
    def _phase_b():
        b = (i - nb) // blk_per_b
        mean = stats_ref[0, b] / n_elem
        var = stats_ref[1, b] / n_elem - mean * mean
        std = jnp.sqrt(var + 1e-5)
        xn = (x - mean) / std
        tid = tid_ref[b]
        tb = temb_ref[pl.ds(tid, 1), :]
        te_row = jnp.dot(tb.astype(jnp.bfloat16),
                         gw_ref[pl.ds(D_MODEL, D_TASK_EMBED), :].astype(jnp.bfloat16),
                         preferred_element_type=jnp.float32) + gb_ref[...]
        logits = jnp.dot(xn.astype(jnp.bfloat16),
                         gw_ref[pl.ds(0, D_MODEL), :].astype(jnp.bfloat16),
                         preferred_element_type=jnp.float32) + te_row
        lg_ref[...] = logits
        ii = lax.broadcasted_iota(jnp.int32, logits.shape, 1)
        m1 = jnp.max(logits, axis=1, keepdims=True)
        i1 = jnp.min(jnp.where(logits == m1, ii, NUM_EXPERTS), axis=1,
                     keepdims=True)
        l2 = jnp.where(ii == i1, -jnp.inf, logits)
        m2 = jnp.max(l2, axis=1, keepdims=True)
        i2 = jnp.min(jnp.where(l2 == m2, ii, NUM_EXPERTS), axis=1,
                     keepdims=True)
        ed = jnp.exp(m2 - m1)
        p1 = 1.0 / (1.0 + ed)
        p2 = ed / (1.0 + ed)
        w_ref[...] = jnp.where(ii == i1, p1, 0.0) + jnp.where(ii == i2, p2, 0.0)


def _route_body(lg_hbm, idxt_hbm, cnt_hbm, lg_v, i1_v, i2_v, cnt_v):
    tok_per_w = lg_hbm.shape[0] // SC_WORKERS
    n_grp = tok_per_w // SC_LANES
    wid = lax.axis_index("s") * SC_CORES + lax.axis_index("c")
    base = wid * tok_per_w
    pltpu.sync_copy(lg_hbm.at[pl.ds(base, tok_per_w), :], lg_v)

    lane = lax.iota(jnp.int32, SC_LANES)
    neg_inf = jnp.full((SC_LANES,), -jnp.inf, jnp.float32)
    cnt = [jnp.zeros((SC_LANES,), jnp.float32) for _ in range(NUM_EXPERTS)]

    for j in range(n_grp):
        row = jnp.full((SC_LANES,), j * SC_LANES, jnp.int32) + lane
        v = [plsc.load_gather(lg_v, [row, jnp.full((SC_LANES,), e, jnp.int32)])
             for e in range(NUM_EXPERTS)]
        m1 = v[0]
        for e in range(1, NUM_EXPERTS):
            m1 = jnp.maximum(m1, v[e])
        i1 = jnp.full((SC_LANES,), NUM_EXPERTS, jnp.int32)
        for e in reversed(range(NUM_EXPERTS)):
            i1 = jnp.where(v[e] == m1, jnp.full((SC_LANES,), e, jnp.int32), i1)
        l2 = [jnp.where(i1 == e, neg_inf, v[e]) for e in range(NUM_EXPERTS)]
        m2 = l2[0]
        for e in range(1, NUM_EXPERTS):
            m2 = jnp.maximum(m2, l2[e])
        i2 = jnp.full((SC_LANES,), NUM_EXPERTS, jnp.int32)
        for e in reversed(range(NUM_EXPERTS)):
            i2 = jnp.where(l2[e] == m2, jnp.full((SC_LANES,), e, jnp.int32), i2)
        zero = jnp.zeros((SC_LANES,), jnp.float32)
        one = jnp.full((SC_LANES,), 1.0, jnp.float32)
        for e in range(NUM_EXPERTS):
            cnt[e] = (cnt[e] + jnp.where(i1 == e, one, zero)
                      + jnp.where(i2 == e, one, zero))
        i1_v[pl.ds(j * SC_LANES, SC_LANES)] = i1
        i2_v[pl.ds(j * SC_LANES, SC_LANES)] = i2

    for e in range(NUM_EXPERTS):
        cnt_v[e, :] = cnt[e]
    pltpu.sync_copy(i1_v, idxt_hbm.at[0, pl.ds(base, tok_per_w)])
    pltpu.sync_copy(i2_v, idxt_hbm.at[1, pl.ds(base, tok_per_w)])
    pltpu.sync_copy(cnt_v, cnt_hbm.at[wid])


def _stage3_body(h_ref, w_ref, w2_ref, b2_ref, out_ref):
    w = w_ref[...]
    exp_mat = jnp.where(
        lax.broadcasted_iota(jnp.int32, (NUM_EXPERTS, EF), 1) // D_FFN
        == lax.broadcasted_iota(jnp.int32, (NUM_EXPERTS, EF), 0),
        1.0, 0.0)
    w_exp = jnp.dot(w, exp_mat, preferred_element_type=jnp.float32)
    hw = (h_ref[...].astype(jnp.float32) * w_exp).astype(jnp.bfloat16)
    out = jnp.dot(hw, w2_ref[...], preferred_element_type=jnp.float32)
    out = out + jnp.dot(w, b2_ref[...], preferred_element_type=jnp.float32)
    out_ref[...] = out


@jax.jit
def kernel(x, task_id, task_emb, gate_W, gate_b, W1, b1, W2, b2):
    B, S, D = x.shape
    T = B * S
    nblk = T // TOK_BLK
    tok_per_w = T // SC_WORKERS

    x2d = x.reshape(T, D)
    w1a = W1.transpose(1, 0, 2).reshape(D, EF).astype(jnp.bfloat16)
    b1f = b1.reshape(1, EF)
    gbr = gate_b.reshape(1, NUM_EXPERTS)
    w2a = W2.reshape(EF, D).astype(jnp.bfloat16)
    tid32 = task_id.astype(jnp.int32)

    h, logits, w_tok = pl.pallas_call(
        _stage12_body,
        grid=(2 * nblk,),
        in_specs=[
            pl.BlockSpec(memory_space=pltpu.MemorySpace.SMEM),
            pl.BlockSpec((TOK_BLK, D),
                         lambda i: (jnp.where(i < nblk, i, i - nblk), 0)),
            pl.BlockSpec((D, EF), lambda i: (0, 0)),
            pl.BlockSpec((1, EF), lambda i: (0, 0)),
            pl.BlockSpec((D + D_TASK_EMBED, NUM_EXPERTS), lambda i: (0, 0)),
            pl.BlockSpec((1, NUM_EXPERTS), lambda i: (0, 0)),
            pl.BlockSpec((NUM_TASKS, D_TASK_EMBED), lambda i: (0, 0)),
        ],
        out_specs=[
            pl.BlockSpec((TOK_BLK, EF), lambda i: (jnp.minimum(i, nblk - 1), 0)),
            pl.BlockSpec((TOK_BLK, NUM_EXPERTS),
                         lambda i: (jnp.maximum(i - nblk, 0), 0)),
            pl.BlockSpec((TOK_BLK, NUM_EXPERTS),
                         lambda i: (jnp.maximum(i - nblk, 0), 0)),
        ],
        out_shape=[
            jax.ShapeDtypeStruct((T, EF), jnp.bfloat16),
            jax.ShapeDtypeStruct((T, NUM_EXPERTS), jnp.float32),
            jax.ShapeDtypeStruct((T, NUM_EXPERTS), jnp.float32),
        ],
        scratch_shapes=[pltpu.SMEM((2, B), jnp.float32)],
    )(tid32, x2d, w1a, b1f, gate_W, gbr, task_emb)

    mesh = plsc.VectorSubcoreMesh(core_axis_name="c", subcore_axis_name="s")
    idxt, cnt = pl.kernel(
        _route_body,
        out_type=[
            jax.ShapeDtypeStruct((TOP_K, T), jnp.int32),
            jax.ShapeDtypeStruct((SC_WORKERS, NUM_EXPERTS, SC_LANES),
                                 jnp.float32),
        ],
        mesh=mesh,
        compiler_params=pltpu.CompilerParams(needs_layout_passes=False),
        scratch_types=[
            pltpu.VMEM((tok_per_w, NUM_EXPERTS), jnp.float32),
            pltpu.VMEM((tok_per_w,), jnp.int32),
            pltpu.VMEM((tok_per_w,), jnp.int32),
            pltpu.VMEM((NUM_EXPERTS, SC_LANES), jnp.float32),
        ],
    )(logits)

    out = pl.pallas_call(
        _stage3_body,
        grid=(nblk,),
        in_specs=[
            pl.BlockSpec((TOK_BLK, EF), lambda i: (i, 0)),
            pl.BlockSpec((TOK_BLK, NUM_EXPERTS), lambda i: (i, 0)),
            pl.BlockSpec((EF, D), lambda i: (0, 0)),
            pl.BlockSpec((NUM_EXPERTS, D), lambda i: (0, 0)),
        ],
        out_specs=pl.BlockSpec((TOK_BLK, D), lambda i: (i, 0)),
        out_shape=jax.ShapeDtypeStruct((T, D), jnp.float32),
    )(h, w_tok, w2a, b2)

    final_output = out.reshape(B, S, D)
    topk_idx = jnp.transpose(idxt, (1, 0)).reshape(B, S, TOP_K)

    counts = jnp.sum(cnt, axis=(0, 2))
    usage_mean = jnp.mean(counts) + 1e-6
    usage_std = jnp.std(counts, ddof=1)
    lb_loss = (usage_std / usage_mean) ** 2
    return (final_output, lb_loss, topk_idx)
